# Initial kernel scaffold; baseline (speedup 1.0000x reference)
import functools

import jax
import jax.numpy as jnp
import numpy as np
from jax import lax
from jax.experimental import pallas as pl
from jax.experimental.pallas import tpu as pltpu
from jax.experimental.pallas import tpu_sc as plsc

N = 10000
E = 320000
H = 128
OUT = 128
EF = 16
NSIG = 15
DTAB = 144
LG = DTAB // 16
CB = 80
BN = 2000
BE = 2000

_info = plsc.get_sparse_core_info()
_NC, _NS = _info.num_cores, _info.num_subcores
NW = _NC * _NS
EPW = E // NW
NCH = EPW // CB


def _tables_body(f_ref, c_ref, ws_ref, wd_ref, ta_ref, tb_ref):
    f = f_ref[...]
    ta_ref[:, :H] = jnp.dot(f, ws_ref[...], preferred_element_type=jnp.float32)
    ta_ref[:, H:] = c_ref[...]
    tb_ref[:, :H] = jnp.dot(f, wd_ref[...], preferred_element_type=jnp.float32)
    tb_ref[:, H:] = -c_ref[...]


def _build_tables(feats, coors_pad, w1s, w1d):
    return pl.pallas_call(
        _tables_body,
        grid=(N // BN,),
        in_specs=[
            pl.BlockSpec((BN, H), lambda i: (i, 0)),
            pl.BlockSpec((BN, 16), lambda i: (i, 0)),
            pl.BlockSpec((H, H), lambda i: (0, 0)),
            pl.BlockSpec((H, H), lambda i: (0, 0)),
        ],
        out_specs=[
            pl.BlockSpec((BN, DTAB), lambda i: (i, 0)),
            pl.BlockSpec((BN, DTAB), lambda i: (i, 0)),
        ],
        out_shape=[
            jax.ShapeDtypeStruct((N, DTAB), jnp.float32),
            jax.ShapeDtypeStruct((N, DTAB), jnp.float32),
        ],
    )(feats, coors_pad, w1s, w1d)


def _gather_body(ta, tb, src, dst, out, idx_s, idx_d, rows_a, rows_b, sem):
    wid = lax.axis_index("s") * _NC + lax.axis_index("c")
    base = wid * EPW

    def chunk(c, carry):
        e0 = base + c * CB
        pltpu.sync_copy(src.at[pl.ds(e0, CB)], idx_s)
        pltpu.sync_copy(dst.at[pl.ds(e0, CB)], idx_d)
        cp_a = pltpu.async_copy(ta.at[idx_s], rows_a, sem)
        cp_b = pltpu.async_copy(tb.at[idx_d], rows_b, sem)
        cp_a.wait()
        cp_b.wait()

        def row(i, carry2):
            for j in range(LG):
                sl = pl.ds(j * 16, 16)
                rows_a[i, sl] = rows_a[i, sl] + rows_b[i, sl]
            return carry2

        lax.fori_loop(0, CB, row, 0)
        pltpu.sync_copy(rows_a, out.at[pl.ds(e0, CB)])
        return carry

    lax.fori_loop(0, NCH, chunk, 0)


_gather = functools.partial(
    pl.kernel,
    out_type=jax.ShapeDtypeStruct((E, DTAB), jnp.float32),
    mesh=plsc.VectorSubcoreMesh(core_axis_name="c", subcore_axis_name="s"),
    scratch_types=[
        pltpu.VMEM((CB,), jnp.int32),
        pltpu.VMEM((CB,), jnp.int32),
        pltpu.VMEM((CB, DTAB), jnp.float32),
        pltpu.VMEM((CB, DTAB), jnp.float32),
        pltpu.SemaphoreType.DMA,
    ],
)(_gather_body)


def _mlp_body(m_ref, ef_ref, w1e_ref, w1m_ref, isg_ref, b1_ref, lng_ref,
              lnb_ref, w2_ref, b2_ref, o_ref):
    m = m_ref[...]
    g = m[:, :OUT]
    xd = m[:, OUT:]
    d2 = jnp.sum(xd * xd, axis=1, keepdims=True)
    mag = jnp.exp(-d2 * isg_ref[...])
    u = (g
         + jnp.dot(ef_ref[...], w1e_ref[...], preferred_element_type=jnp.float32)
         + jnp.dot(mag, w1m_ref[...], preferred_element_type=jnp.float32)
         + b1_ref[...])
    h = jnp.where(u > 0, u, 0.01 * u)
    mu = jnp.mean(h, axis=-1, keepdims=True)
    hc = h - mu
    var = jnp.mean(hc * hc, axis=-1, keepdims=True)
    hn = hc * lax.rsqrt(var + 1e-5) * lng_ref[...] + lnb_ref[...]
    o_ref[...] = jnp.dot(hn, w2_ref[...], preferred_element_type=jnp.float32) + b2_ref[...]


def _mlp(m, ef, w1e, w1m, isg, b1, lng, lnb, w2, b2):
    full = lambda i: (0, 0)
    return pl.pallas_call(
        _mlp_body,
        grid=(E // BE,),
        in_specs=[
            pl.BlockSpec((BE, DTAB), lambda i: (i, 0)),
            pl.BlockSpec((BE, EF), lambda i: (i, 0)),
            pl.BlockSpec((EF, OUT), full),
            pl.BlockSpec((16, OUT), full),
            pl.BlockSpec((1, 16), full),
            pl.BlockSpec((1, OUT), full),
            pl.BlockSpec((1, OUT), full),
            pl.BlockSpec((1, OUT), full),
            pl.BlockSpec((OUT, OUT), full),
            pl.BlockSpec((1, OUT), full),
        ],
        out_specs=pl.BlockSpec((BE, OUT), lambda i: (i, 0)),
        out_shape=jax.ShapeDtypeStruct((E, OUT), jnp.float32),
    )(m, ef, w1e, w1m, isg, b1, lng, lnb, w2, b2)


def kernel(coors_ligand, h_feats_ligand, original_ligand_node_features,
           original_edge_feats_ligand, orig_coors_ligand, coors_receptor,
           h_feats_receptor, original_receptor_node_features,
           original_edge_feats_receptor, orig_coors_receptor,
           edge_index_ligand, edge_index_receptor, W1, b1, ln_g, ln_b, W2, b2):
    w1s = W1[:H]
    w1d = W1[H:2 * H]
    w1e = W1[2 * H:2 * H + EF]
    w1m = jnp.concatenate([W1[2 * H + EF:], jnp.zeros((1, OUT), jnp.float32)], axis=0)
    isg = jnp.asarray(
        np.concatenate([1.0 / (1.5 ** np.arange(NSIG)), [0.0]]), jnp.float32
    ).reshape(1, 16)
    b1r = b1.reshape(1, OUT)
    b2r = b2.reshape(1, OUT)
    lngr = ln_g.reshape(1, OUT)
    lnbr = ln_b.reshape(1, OUT)

    cl = jnp.pad(coors_ligand, ((0, 0), (0, 13)))
    cr = jnp.pad(coors_receptor, ((0, 0), (0, 13)))

    ta_l, tb_l = _build_tables(h_feats_ligand, cl, w1s, w1d)
    ta_r, tb_r = _build_tables(h_feats_receptor, cr, w1s, w1d)

    m_l = _gather(ta_l, tb_l, edge_index_ligand[0], edge_index_ligand[1])
    m_r = _gather(ta_r, tb_r, edge_index_receptor[0], edge_index_receptor[1])

    msg_ll = _mlp(m_l, original_edge_feats_ligand, w1e, w1m, isg, b1r, lngr, lnbr, W2, b2r)
    msg_rr = _mlp(m_r, original_edge_feats_receptor, w1e, w1m, isg, b1r, lngr, lnbr, W2, b2r)
    return (msg_ll, msg_rr)


# speedup vs baseline: 4.2810x; 4.2810x over previous
import functools

import jax
import jax.numpy as jnp
import numpy as np
from jax import lax
from jax.experimental import pallas as pl
from jax.experimental.pallas import tpu as pltpu
from jax.experimental.pallas import tpu_sc as plsc

N = 10000
E = 320000
H = 128
OUT = 128
EF = 16
NSIG = 15
CB = 80
BN = 2000
BE = 2000


def _tables_body(f_ref, ws_ref, wd_ref, ta_ref, tb_ref):
    f = f_ref[...]
    ta_ref[...] = jnp.dot(f, ws_ref[...], preferred_element_type=jnp.float32)
    tb_ref[...] = jnp.dot(f, wd_ref[...], preferred_element_type=jnp.float32)


def _build_tables(feats, w1s, w1d):
    return pl.pallas_call(
        _tables_body,
        grid=(N // BN,),
        in_specs=[
            pl.BlockSpec((BN, H), lambda i: (i, 0)),
            pl.BlockSpec((H, H), lambda i: (0, 0)),
            pl.BlockSpec((H, H), lambda i: (0, 0)),
        ],
        out_specs=[
            pl.BlockSpec((BN, H), lambda i: (i, 0)),
            pl.BlockSpec((BN, H), lambda i: (i, 0)),
        ],
        out_shape=[
            jax.ShapeDtypeStruct((N, H), jnp.float32),
            jax.ShapeDtypeStruct((N, H), jnp.float32),
        ],
    )(feats, w1s, w1d)


@functools.lru_cache(maxsize=None)
def _make_gather():
    info = plsc.get_sparse_core_info()
    nc, ns = info.num_cores, info.num_subcores
    nw = nc * ns
    epw = E // nw
    nch = epw // CB

    def _gather_body(ta, tb, c4, src, dst, g_out, d2_out,
                     idx_s, idx_d, rows_a, rows_b, d2_v, c4_v, sem):
        wid = lax.axis_index("s") * nc + lax.axis_index("c")
        base = wid * epw
        pltpu.sync_copy(c4, c4_v)

        def chunk(c, carry):
            e0 = base + c * CB
            pltpu.sync_copy(src.at[pl.ds(e0, CB)], idx_s)
            pltpu.sync_copy(dst.at[pl.ds(e0, CB)], idx_d)
            cp_a = pltpu.async_copy(ta.at[idx_s], rows_a, sem)
            cp_b = pltpu.async_copy(tb.at[idx_d], rows_b, sem)

            for gblk in range(CB // 16):
                sl = pl.ds(gblk * 16, 16)
                is4 = idx_s[sl] << 2
                id4 = idx_d[sl] << 2
                dx = (plsc.load_gather(c4_v, [is4])
                      - plsc.load_gather(c4_v, [id4]))
                dy = (plsc.load_gather(c4_v, [is4 + 1])
                      - plsc.load_gather(c4_v, [id4 + 1]))
                dz = (plsc.load_gather(c4_v, [is4 + 2])
                      - plsc.load_gather(c4_v, [id4 + 2]))
                d2_v[sl] = dx * dx + dy * dy + dz * dz

            cp_a.wait()
            cp_b.wait()

            def row(i, carry2):
                for j in range(H // 16):
                    sl = pl.ds(j * 16, 16)
                    rows_a[i, sl] = rows_a[i, sl] + rows_b[i, sl]
                return carry2

            lax.fori_loop(0, CB, row, 0)
            pltpu.sync_copy(rows_a, g_out.at[pl.ds(e0, CB)])
            pltpu.sync_copy(d2_v, d2_out.at[pl.ds(e0, CB)])
            return carry

        lax.fori_loop(0, nch, chunk, 0)

    return functools.partial(
        pl.kernel,
        out_type=[
            jax.ShapeDtypeStruct((E, H), jnp.float32),
            jax.ShapeDtypeStruct((E,), jnp.float32),
        ],
        mesh=plsc.VectorSubcoreMesh(core_axis_name="c", subcore_axis_name="s"),
        compiler_params=pltpu.CompilerParams(needs_layout_passes=False),
        scratch_types=[
            pltpu.VMEM((CB,), jnp.int32),
            pltpu.VMEM((CB,), jnp.int32),
            pltpu.VMEM((CB, H), jnp.float32),
            pltpu.VMEM((CB, H), jnp.float32),
            pltpu.VMEM((CB,), jnp.float32),
            pltpu.VMEM((4 * N,), jnp.float32),
            pltpu.SemaphoreType.DMA,
        ],
    )(_gather_body)


def _mlp_body(g_ref, d2_ref, ef_ref, w1e_ref, w1m_ref, isg_ref, b1_ref,
              lng_ref, lnb_ref, w2_ref, b2_ref, o_ref):
    d2 = d2_ref[...]
    mag = jnp.exp(-d2 * isg_ref[...])
    u = (g_ref[...]
         + jnp.dot(ef_ref[...], w1e_ref[...], preferred_element_type=jnp.float32)
         + jnp.dot(mag, w1m_ref[...], preferred_element_type=jnp.float32)
         + b1_ref[...])
    h = jnp.where(u > 0, u, 0.01 * u)
    mu = jnp.mean(h, axis=-1, keepdims=True)
    hc = h - mu
    var = jnp.mean(hc * hc, axis=-1, keepdims=True)
    hn = hc * lax.rsqrt(var + 1e-5) * lng_ref[...] + lnb_ref[...]
    o_ref[...] = jnp.dot(hn, w2_ref[...], preferred_element_type=jnp.float32) + b2_ref[...]


def _mlp(g, d2, ef, w1e, w1m, isg, b1, lng, lnb, w2, b2):
    full = lambda i: (0, 0)
    return pl.pallas_call(
        _mlp_body,
        grid=(E // BE,),
        in_specs=[
            pl.BlockSpec((BE, H), lambda i: (i, 0)),
            pl.BlockSpec((BE, 1), lambda i: (i, 0)),
            pl.BlockSpec((BE, EF), lambda i: (i, 0)),
            pl.BlockSpec((EF, OUT), full),
            pl.BlockSpec((16, OUT), full),
            pl.BlockSpec((1, 16), full),
            pl.BlockSpec((1, OUT), full),
            pl.BlockSpec((1, OUT), full),
            pl.BlockSpec((1, OUT), full),
            pl.BlockSpec((OUT, OUT), full),
            pl.BlockSpec((1, OUT), full),
        ],
        out_specs=pl.BlockSpec((BE, OUT), lambda i: (i, 0)),
        out_shape=jax.ShapeDtypeStruct((E, OUT), jnp.float32),
    )(g, d2, ef, w1e, w1m, isg, b1, lng, lnb, w2, b2)


def kernel(coors_ligand, h_feats_ligand, original_ligand_node_features,
           original_edge_feats_ligand, orig_coors_ligand, coors_receptor,
           h_feats_receptor, original_receptor_node_features,
           original_edge_feats_receptor, orig_coors_receptor,
           edge_index_ligand, edge_index_receptor, W1, b1, ln_g, ln_b, W2, b2):
    w1s = W1[:H]
    w1d = W1[H:2 * H]
    w1e = W1[2 * H:2 * H + EF]
    w1m = jnp.concatenate([W1[2 * H + EF:], jnp.zeros((1, OUT), jnp.float32)], axis=0)
    isg = jnp.asarray(
        np.concatenate([1.0 / (1.5 ** np.arange(NSIG)), [0.0]]), jnp.float32
    ).reshape(1, 16)
    b1r = b1.reshape(1, OUT)
    b2r = b2.reshape(1, OUT)
    lngr = ln_g.reshape(1, OUT)
    lnbr = ln_b.reshape(1, OUT)

    c4_l = jnp.pad(coors_ligand, ((0, 0), (0, 1))).reshape(4 * N)
    c4_r = jnp.pad(coors_receptor, ((0, 0), (0, 1))).reshape(4 * N)

    ta_l, tb_l = _build_tables(h_feats_ligand, w1s, w1d)
    ta_r, tb_r = _build_tables(h_feats_receptor, w1s, w1d)

    gather = _make_gather()
    g_l, d2_l = gather(ta_l, tb_l, c4_l, edge_index_ligand[0], edge_index_ligand[1])
    g_r, d2_r = gather(ta_r, tb_r, c4_r, edge_index_receptor[0], edge_index_receptor[1])

    msg_ll = _mlp(g_l, d2_l.reshape(E, 1), original_edge_feats_ligand,
                  w1e, w1m, isg, b1r, lngr, lnbr, W2, b2r)
    msg_rr = _mlp(g_r, d2_r.reshape(E, 1), original_edge_feats_receptor,
                  w1e, w1m, isg, b1r, lngr, lnbr, W2, b2r)
    return (msg_ll, msg_rr)
